# baseline (speedup 1.0000x reference)
import functools

import jax
import jax.numpy as jnp
import numpy as np
from jax import lax
from jax.experimental import pallas as pl
from jax.experimental.pallas import tpu as pltpu
from jax.experimental.pallas import tpu_sc as plsc

B = 4096
F = 26
V = 100000
D = 32
IN_DIM = F * D
PAD_DIM = 896
NU = PAD_DIM // 128
HID = 400
BF = B * F

OUT_ROWS = B * PAD_DIM // D
XP_ROWS = B * PAD_DIM // 128


_NC = 2
_NS = 16
_NW = _NC * _NS
_ROWS_W = BF // _NW


def _sc_gather(idx_hbm, oidx_hbm, table_hbm, out_hbm, idx_v, oidx_v, rows_v,
               sem):
    wid = lax.axis_index("s") * _NC + lax.axis_index("c")
    base = wid * _ROWS_W
    pltpu.sync_copy(idx_hbm.at[pl.ds(base, _ROWS_W)], idx_v)
    pltpu.sync_copy(oidx_hbm.at[pl.ds(base, _ROWS_W)], oidx_v)
    pltpu.async_copy(table_hbm.at[idx_v], rows_v, sem).wait()
    pltpu.async_copy(rows_v, out_hbm.at[oidx_v], sem).wait()


def _gather_rows(idx_flat, out_idx, table):
    mesh = plsc.VectorSubcoreMesh(core_axis_name="c", subcore_axis_name="s")
    k = functools.partial(
        pl.kernel,
        mesh=mesh,
        compiler_params=pltpu.CompilerParams(use_tc_tiling_on_sc=False),
        out_type=jax.ShapeDtypeStruct((OUT_ROWS, D), jnp.float32),
        scratch_types=[
            pltpu.VMEM((_ROWS_W,), jnp.int32),
            pltpu.VMEM((_ROWS_W,), jnp.int32),
            pltpu.VMEM((_ROWS_W, D), jnp.float32),
            pltpu.SemaphoreType.DMA,
        ],
    )(_sc_gather)
    return k(idx_flat, out_idx, table)


_BLK = 512
_NB = B // _BLK
_TB = _BLK // 8
_XPB = _BLK * PAD_DIM // 128


def _dense_body(xp_ref, vals_ref, e_ref, w1_ref, b1_ref, w2_ref, b2_ref,
                w3_ref, b3_ref, cw_ref, cb_ref, wdh_ref, wdx_ref, bd_ref,
                out_ref):
    f32 = jnp.float32
    xb = xp_ref[...].reshape(_TB, NU, 8, 128)
    scale = jnp.dot(vals_ref[...], e_ref[...], preferred_element_type=f32)
    lane = lax.broadcasted_iota(jnp.int32, (_BLK, 128), 1)
    x0 = []
    for u in range(NU):
        g = xb[:, u].reshape(_BLK, 128)
        if u == NU - 1:
            g = jnp.where(lane < 64, g, 0.0)
        x0.append(g * scale[:, u * 128:(u + 1) * 128])
    bf16 = jnp.bfloat16
    h = b1_ref[...]
    for u in range(NU):
        h = h + jnp.dot(x0[u].astype(bf16), w1_ref[u * 128:(u + 1) * 128, :],
                        preferred_element_type=f32)
    h = jnp.maximum(h, 0.0)
    h = jnp.maximum(jnp.dot(h.astype(bf16), w2_ref[...],
                            preferred_element_type=f32) + b2_ref[...], 0.0)
    h = jnp.maximum(jnp.dot(h.astype(bf16), w3_ref[...],
                            preferred_element_type=f32) + b3_ref[...], 0.0)
    xc = list(x0)
    for i in range(3):
        xw = jnp.zeros((_BLK, 1), f32)
        for u in range(NU):
            xw = xw + jnp.sum(xc[u] * cw_ref[i, u * 128:(u + 1) * 128][None, :],
                              axis=1, keepdims=True)
        for u in range(NU):
            xc[u] = x0[u] * xw + cb_ref[i, u * 128:(u + 1) * 128][None, :] \
                + xc[u]
    logits = jnp.dot(h, wdh_ref[...], preferred_element_type=f32) + bd_ref[...]
    for u in range(NU):
        logits = logits + jnp.dot(xc[u], wdx_ref[u * 128:(u + 1) * 128, :],
                                  preferred_element_type=f32)
    out_ref[...] = jax.nn.sigmoid(logits)


def _dense(xp, vals, expand, W1p, b1, W2, b2, W3, b3, cwp, cbp, Wdh, Wdxp,
           bd):
    full2 = lambda shape: pl.BlockSpec(shape, lambda i: (0, 0))
    return pl.pallas_call(
        _dense_body,
        grid=(_NB,),
        in_specs=[
            pl.BlockSpec((_XPB, 128), lambda i: (i, 0)),
            pl.BlockSpec((_BLK, F), lambda i: (i, 0)),
            full2((F, PAD_DIM)),
            full2((PAD_DIM, HID)),
            full2((1, HID)),
            full2((HID, HID)),
            full2((1, HID)),
            full2((HID, HID)),
            full2((1, HID)),
            full2((3, PAD_DIM)),
            full2((3, PAD_DIM)),
            full2((HID, 1)),
            full2((PAD_DIM, 1)),
            full2((1, 1)),
        ],
        out_specs=pl.BlockSpec((_BLK, 1), lambda i: (i, 0)),
        out_shape=jax.ShapeDtypeStruct((B, 1), jnp.float32),
    )(xp, vals, expand, W1p, b1, W2, b2, W3, b3, cwp, cbp, Wdh, Wdxp, bd)


def _pad_cols(a, n):
    return jnp.concatenate(
        [a, jnp.zeros(a.shape[:-1] + (n - a.shape[-1],), a.dtype)], axis=-1)


def kernel(feature_idx, feature_vals, feature_embedding, W1, b1, W2, b2, W3,
           b3, cw0, cb0, cw1, cb1, cw2, cb2, Wd, bd):
    idx_flat = feature_idx.reshape(BF)
    p = np.arange(BF, dtype=np.int64)
    b_, f_ = p // F, p % F
    out_idx = jnp.asarray(
        ((b_ // 8) * NU + f_ // 4) * 32 + (b_ % 8) * 4 + (f_ % 4),
        dtype=jnp.int32)
    gathered = _gather_rows(idx_flat, out_idx, feature_embedding)
    xp = gathered.reshape(XP_ROWS, 128)
    e_np = np.zeros((F, PAD_DIM), dtype=np.float32)
    for f in range(F):
        e_np[f, f * D:(f + 1) * D] = 1.0
    expand = jnp.asarray(e_np)
    W1p = jnp.concatenate(
        [W1, jnp.zeros((PAD_DIM - IN_DIM, HID), jnp.float32)],
        axis=0).astype(jnp.bfloat16)
    cwp = _pad_cols(jnp.stack([cw0, cw1, cw2]), PAD_DIM)
    cbp = _pad_cols(jnp.stack([cb0, cb1, cb2]), PAD_DIM)
    Wdxp = jnp.concatenate(
        [Wd[HID:], jnp.zeros((PAD_DIM - IN_DIM, 1), jnp.float32)], axis=0)
    return _dense(xp, feature_vals, expand, W1p, b1.reshape(1, HID),
                  W2.astype(jnp.bfloat16), b2.reshape(1, HID),
                  W3.astype(jnp.bfloat16), b3.reshape(1, HID), cwp, cbp,
                  Wd[:HID], Wdxp, bd.reshape(1, 1))


# speedup vs baseline: 1.0014x; 1.0014x over previous
import functools

import jax
import jax.numpy as jnp
import numpy as np
from jax import lax
from jax.experimental import pallas as pl
from jax.experimental.pallas import tpu as pltpu
from jax.experimental.pallas import tpu_sc as plsc

B = 4096
F = 26
V = 100000
D = 32
IN_DIM = F * D
PAD_DIM = 896
NU = PAD_DIM // 128
HID = 400
BF = B * F

OUT_ROWS = B * PAD_DIM // D
XP_ROWS = B * PAD_DIM // 128


_NC = 2
_NS = 16
_NW = _NC * _NS
_ROWS_W = BF // _NW


def _sc_gather(idx_hbm, oidx_hbm, table_hbm, out_hbm, idx_v, oidx_v, rows_v,
               sem):
    wid = lax.axis_index("s") * _NC + lax.axis_index("c")
    base = wid * _ROWS_W
    pltpu.sync_copy(idx_hbm.at[pl.ds(base, _ROWS_W)], idx_v)
    pltpu.sync_copy(oidx_hbm.at[pl.ds(base, _ROWS_W)], oidx_v)
    pltpu.async_copy(table_hbm.at[idx_v], rows_v, sem).wait()
    pltpu.async_copy(rows_v, out_hbm.at[oidx_v], sem).wait()


def _gather_rows(idx_flat, out_idx, table):
    mesh = plsc.VectorSubcoreMesh(core_axis_name="c", subcore_axis_name="s")
    k = functools.partial(
        pl.kernel,
        mesh=mesh,
        compiler_params=pltpu.CompilerParams(use_tc_tiling_on_sc=False),
        out_type=jax.ShapeDtypeStruct((OUT_ROWS, D), jnp.float32),
        scratch_types=[
            pltpu.VMEM((_ROWS_W,), jnp.int32),
            pltpu.VMEM((_ROWS_W,), jnp.int32),
            pltpu.VMEM((_ROWS_W, D), jnp.float32),
            pltpu.SemaphoreType.DMA,
        ],
    )(_sc_gather)
    return k(idx_flat, out_idx, table)


_BLK = 1024
_NB = B // _BLK
_TB = _BLK // 8
_XPB = _BLK * PAD_DIM // 128


def _dense_body(xp_ref, vals_ref, e_ref, w1_ref, b1_ref, w2_ref, b2_ref,
                w3_ref, b3_ref, cw_ref, cb_ref, wdh_ref, wdx_ref, bd_ref,
                out_ref):
    f32 = jnp.float32
    xb = xp_ref[...].reshape(_TB, NU, 8, 128)
    scale = jnp.dot(vals_ref[...], e_ref[...], preferred_element_type=f32)
    lane = lax.broadcasted_iota(jnp.int32, (_BLK, 128), 1)
    x0 = []
    for u in range(NU):
        g = xb[:, u].reshape(_BLK, 128)
        if u == NU - 1:
            g = jnp.where(lane < 64, g, 0.0)
        x0.append(g * scale[:, u * 128:(u + 1) * 128])
    bf16 = jnp.bfloat16
    h = b1_ref[...]
    for u in range(NU):
        h = h + jnp.dot(x0[u].astype(bf16), w1_ref[u * 128:(u + 1) * 128, :],
                        preferred_element_type=f32)
    h = jnp.maximum(h, 0.0)
    h = jnp.maximum(jnp.dot(h.astype(bf16), w2_ref[...],
                            preferred_element_type=f32) + b2_ref[...], 0.0)
    h = jnp.maximum(jnp.dot(h.astype(bf16), w3_ref[...],
                            preferred_element_type=f32) + b3_ref[...], 0.0)
    xc = list(x0)
    for i in range(3):
        xw = jnp.zeros((_BLK, 1), f32)
        for u in range(NU):
            xw = xw + jnp.sum(xc[u] * cw_ref[i, u * 128:(u + 1) * 128][None, :],
                              axis=1, keepdims=True)
        for u in range(NU):
            xc[u] = x0[u] * xw + cb_ref[i, u * 128:(u + 1) * 128][None, :] \
                + xc[u]
    logits = jnp.dot(h, wdh_ref[...], preferred_element_type=f32) + bd_ref[...]
    for u in range(NU):
        logits = logits + jnp.dot(xc[u], wdx_ref[u * 128:(u + 1) * 128, :],
                                  preferred_element_type=f32)
    out_ref[...] = jax.nn.sigmoid(logits)


def _dense(xp, vals, expand, W1p, b1, W2, b2, W3, b3, cwp, cbp, Wdh, Wdxp,
           bd):
    full2 = lambda shape: pl.BlockSpec(shape, lambda i: (0, 0))
    return pl.pallas_call(
        _dense_body,
        grid=(_NB,),
        in_specs=[
            pl.BlockSpec((_XPB, 128), lambda i: (i, 0)),
            pl.BlockSpec((_BLK, F), lambda i: (i, 0)),
            full2((F, PAD_DIM)),
            full2((PAD_DIM, HID)),
            full2((1, HID)),
            full2((HID, HID)),
            full2((1, HID)),
            full2((HID, HID)),
            full2((1, HID)),
            full2((3, PAD_DIM)),
            full2((3, PAD_DIM)),
            full2((HID, 1)),
            full2((PAD_DIM, 1)),
            full2((1, 1)),
        ],
        out_specs=pl.BlockSpec((_BLK, 1), lambda i: (i, 0)),
        out_shape=jax.ShapeDtypeStruct((B, 1), jnp.float32),
    )(xp, vals, expand, W1p, b1, W2, b2, W3, b3, cwp, cbp, Wdh, Wdxp, bd)


def _pad_cols(a, n):
    return jnp.concatenate(
        [a, jnp.zeros(a.shape[:-1] + (n - a.shape[-1],), a.dtype)], axis=-1)


def kernel(feature_idx, feature_vals, feature_embedding, W1, b1, W2, b2, W3,
           b3, cw0, cb0, cw1, cb1, cw2, cb2, Wd, bd):
    idx_flat = feature_idx.reshape(BF)
    p = np.arange(BF, dtype=np.int64)
    b_, f_ = p // F, p % F
    out_idx = jnp.asarray(
        ((b_ // 8) * NU + f_ // 4) * 32 + (b_ % 8) * 4 + (f_ % 4),
        dtype=jnp.int32)
    gathered = _gather_rows(idx_flat, out_idx, feature_embedding)
    xp = gathered.reshape(XP_ROWS, 128)
    e_np = np.zeros((F, PAD_DIM), dtype=np.float32)
    for f in range(F):
        e_np[f, f * D:(f + 1) * D] = 1.0
    expand = jnp.asarray(e_np)
    W1p = jnp.concatenate(
        [W1, jnp.zeros((PAD_DIM - IN_DIM, HID), jnp.float32)],
        axis=0).astype(jnp.bfloat16)
    cwp = _pad_cols(jnp.stack([cw0, cw1, cw2]), PAD_DIM)
    cbp = _pad_cols(jnp.stack([cb0, cb1, cb2]), PAD_DIM)
    Wdxp = jnp.concatenate(
        [Wd[HID:], jnp.zeros((PAD_DIM - IN_DIM, 1), jnp.float32)], axis=0)
    return _dense(xp, feature_vals, expand, W1p, b1.reshape(1, HID),
                  W2.astype(jnp.bfloat16), b2.reshape(1, HID),
                  W3.astype(jnp.bfloat16), b3.reshape(1, HID), cwp, cbp,
                  Wd[:HID], Wdxp, bd.reshape(1, 1))
